# baseline (speedup 1.0000x reference)
import functools

import jax
import jax.numpy as jnp
from jax.experimental import pallas as pl
from jax.experimental.pallas import tpu as pltpu

_S, _B, _C = 7, 2, 20
_D = _B * 5 + _C
_CELLS = _S * _S
_LAMBDA_COORD, _LAMBDA_NOOBJ = 5.0, 0.5
_L2CLAMP = -144.26950408889634
_CHUNK = 512


def _reduce_weights():
    r10 = jax.lax.broadcasted_iota(jnp.int32, (8, 10), 0)
    k10 = jax.lax.broadcasted_iota(jnp.int32, (8, 10), 1)
    wsq = ((r10 == 0) & (k10 < 4)) | ((r10 == 1) & (k10 >= 5) & (k10 < 9))
    wsq = wsq | ((r10 == 2) & ((k10 == 4) | (k10 == 9)))
    r20 = jax.lax.broadcasted_iota(jnp.int32, (8, _C), 0)
    return wsq.astype(jnp.bfloat16), (r20 == 0).astype(jnp.bfloat16)


def _loss_body(p_ref, t_ref, o_ref):
    j = pl.program_id(0)
    wsq, wbce = _reduce_weights()

    @pl.when(j == 0)
    def _zero():
        o_ref[...] = jnp.zeros_like(o_ref)

    for l in range(0, o_ref.shape[2], _CHUNK):
        sl = pl.ds(l, _CHUNK)

        t05 = t_ref[0, 0:5, sl]
        tbar = jnp.concatenate([t05, t05], axis=0)
        diff = p_ref[0, 0:10, sl] - tbar
        red = jax.lax.dot_general(
            wsq, (diff * diff).astype(jnp.bfloat16), (((1,), (0,)), ((), ())),
            preferred_element_type=jnp.float32,
        )
        d0 = red[0:1]
        d1 = red[1:2]
        sq01 = red[2:3]

        tw = t_ref[0, 2:3, sl]
        th = t_ref[0, 3:4, sl]
        tconf = t_ref[0, 4:5, sl]
        pw0 = p_ref[0, 2:3, sl]
        ph0 = p_ref[0, 3:4, sl]
        pw1 = p_ref[0, 7:8, sl]
        ph1 = p_ref[0, 8:9, sl]
        i0 = jnp.minimum(pw0, tw) * jnp.minimum(ph0, th)
        i1 = jnp.minimum(pw1, tw) * jnp.minimum(ph1, th)
        tae = tw * th + 1e-6
        u0 = pw0 * ph0 + tae - i0
        u1 = pw1 * ph1 + tae - i1
        swap = i1 * u0 > i0 * u1

        xc = p_ref[0, 10:30, sl]
        yc = t_ref[0, 10:30, sl]
        lg = jnp.maximum(jnp.log2(xc), _L2CLAMP)
        l1 = jnp.maximum(jnp.log2(1.0 - xc), _L2CLAMP)
        bfield = (yc * (lg - l1) + l1).astype(jnp.bfloat16)
        bpos = jax.lax.dot_general(
            wbce, bfield, (((1,), (0,)), ((), ())),
            preferred_element_type=jnp.float32,
        )[0:1]

        coordrow = tconf * jnp.where(swap, d1, d0)
        objrow = tconf * sq01
        noobjrow = sq01 - objrow
        classrow = tconf * bpos

        o_ref[0, 0:1, sl] += coordrow
        o_ref[0, 1:2, sl] += objrow
        o_ref[0, 2:3, sl] += noobjrow
        o_ref[0, 3:4, sl] += classrow


@jax.jit
def _yolo_loss(predictions, targets):
    n = predictions.shape[0]
    p = jnp.transpose(predictions, (1, 2, 3, 0)).reshape(_CELLS, _D, n)
    t = jnp.transpose(targets, (1, 2, 3, 0)).reshape(_CELLS, _D, n)
    partial = pl.pallas_call(
        _loss_body,
        grid=(_CELLS,),
        in_specs=[
            pl.BlockSpec((1, _D, n), lambda j: (j, 0, 0)),
            pl.BlockSpec((1, _D, n), lambda j: (j, 0, 0)),
        ],
        out_specs=pl.BlockSpec((1, 4, n), lambda j: (0, 0, 0)),
        out_shape=jax.ShapeDtypeStruct((1, 4, n), jnp.float32),
        compiler_params=pltpu.CompilerParams(
            dimension_semantics=("arbitrary",),
        ),
    )(p, t)
    sums = jnp.sum(partial, axis=(0, 2))
    ln2 = 0.6931471805599453
    coord, objl, nobjl, classl = sums[0], sums[1], sums[2], -ln2 * sums[3]
    total = (_LAMBDA_COORD * coord + objl + _LAMBDA_NOOBJ * nobjl + classl) / n
    return (total, coord / n, objl / n, nobjl / n, classl / n)


def kernel(predictions, targets):
    return _yolo_loss(predictions, targets)
